# Initial kernel scaffold; baseline (speedup 1.0000x reference)
import jax
import jax.numpy as jnp
from jax.experimental import pallas as pl


def kernel(x, h, edge_index, edge_type, edge_weight, W_rel, W_root, bias, W_skip, b_skip):
    raise NotImplementedError("write your pallas kernel here")


# speedup vs baseline: 16.8424x; 16.8424x over previous
import functools

import jax
import jax.numpy as jnp
from jax import lax
from jax.experimental import pallas as pl
from jax.experimental.pallas import tpu as pltpu
from jax.experimental.pallas import tpu_sc as plsc

N = 10000
E = 320000
D = 128
O = 128
R = 8

NC = 2
NS = 16
NW = NC * NS
EPT = E // NW
CH = 80
NCH = EPT // CH
LANES = 16

ROWS_A = 624
ROWS_LAST = N - (NS - 1) * ROWS_A

BN = 2000
NB = N // BN


def _dense_body(h_ref, x_ref, wrel_ref, wroot_ref, wskip_ref, bias_ref,
                bskip_ref, hrel_ref, base_ref):
    r = pl.program_id(1)
    hblk = h_ref[...]
    hrel_ref[...] = jnp.dot(hblk, wrel_ref[0],
                            preferred_element_type=jnp.float32)

    @pl.when(r == 0)
    def _():
        skip = lax.dot_general(x_ref[...], wskip_ref[...],
                               (((1,), (1,)), ((), ())),
                               preferred_element_type=jnp.float32)
        root = jnp.dot(hblk, wroot_ref[...],
                       preferred_element_type=jnp.float32)
        base_ref[...] = skip + root + bias_ref[...] + bskip_ref[...]


_dense = pl.pallas_call(
    _dense_body,
    grid=(NB, R),
    in_specs=[
        pl.BlockSpec((BN, D), lambda nb, r: (nb, 0)),
        pl.BlockSpec((BN, D), lambda nb, r: (nb, 0)),
        pl.BlockSpec((1, D, O), lambda nb, r: (r, 0, 0)),
        pl.BlockSpec((D, O), lambda nb, r: (0, 0)),
        pl.BlockSpec((O, D), lambda nb, r: (0, 0)),
        pl.BlockSpec((1, O), lambda nb, r: (0, 0)),
        pl.BlockSpec((1, O), lambda nb, r: (0, 0)),
    ],
    out_specs=[
        pl.BlockSpec((BN, O), lambda nb, r: (r * NB + nb, 0)),
        pl.BlockSpec((BN, O), lambda nb, r: (nb, 0)),
    ],
    out_shape=[
        jax.ShapeDtypeStruct((R * N, O), jnp.float32),
        jax.ShapeDtypeStruct((N, O), jnp.float32),
    ],
)


def _idx_body(et_ref, src_ref, o_ref):
    o_ref[...] = et_ref[...] * N + src_ref[...]


_idx = pl.pallas_call(
    _idx_body,
    out_shape=jax.ShapeDtypeStruct((E // 128, 128), jnp.int32),
)


def _sc_edges_body(idx_hbm, ew_hbm, dst_hbm, hrel_hbm, zeros_hbm,
                   out_hbm, idx_v, dst_v, w_v, rows_v, acc_s, sem):
    c = lax.axis_index("c")
    s = lax.axis_index("s")
    wid = c * NS + s

    @pl.when(s < NS - 1)
    def _():
        pltpu.sync_copy(zeros_hbm.at[pl.ds(s * ROWS_A, ROWS_A)],
                        acc_s.at[pl.ds(s * ROWS_A, ROWS_A)])

    @pl.when(s == NS - 1)
    def _():
        pltpu.sync_copy(zeros_hbm.at[pl.ds((NS - 1) * ROWS_A, ROWS_LAST)],
                        acc_s.at[pl.ds((NS - 1) * ROWS_A, ROWS_LAST)])

    pltpu.sync_copy(idx_hbm.at[wid], idx_v)
    pltpu.sync_copy(ew_hbm.at[wid], w_v)
    pltpu.sync_copy(dst_hbm.at[wid], dst_v)

    plsc.subcore_barrier()

    def chunk(ck, carry):
        cb = ck * CH
        pltpu.async_copy(hrel_hbm.at[idx_v.at[pl.ds(cb, CH)]],
                         rows_v, sem).wait()

        def scale(g, carry2):
            wv = w_v[pl.ds(cb + g * LANES, LANES)]
            for j in range(LANES):
                e = g * LANES + j
                w = wv[j]
                for c16 in range(O // LANES):
                    sl = pl.ds(c16 * LANES, LANES)
                    rows_v[e, sl] = rows_v[e, sl] * w
            return carry2

        lax.fori_loop(0, CH // LANES, scale, 0)
        pltpu.sync_copy(rows_v, acc_s.at[dst_v.at[ck]], add=True)
        return carry

    lax.fori_loop(0, NCH, chunk, 0)

    plsc.subcore_barrier()

    @pl.when(s < NS - 1)
    def _():
        pltpu.sync_copy(acc_s.at[pl.ds(s * ROWS_A, ROWS_A)],
                        out_hbm.at[c, pl.ds(s * ROWS_A, ROWS_A)])

    @pl.when(s == NS - 1)
    def _():
        pltpu.sync_copy(acc_s.at[pl.ds((NS - 1) * ROWS_A, ROWS_LAST)],
                        out_hbm.at[c, pl.ds((NS - 1) * ROWS_A, ROWS_LAST)])


_sc_edges = functools.partial(
    pl.kernel,
    mesh=plsc.VectorSubcoreMesh(core_axis_name="c", subcore_axis_name="s"),
    out_type=jax.ShapeDtypeStruct((NC, N, O), jnp.float32),
    scratch_types=[
        pltpu.VMEM((EPT,), jnp.int32),
        pltpu.VMEM((NCH, CH), jnp.int32),
        pltpu.VMEM((EPT,), jnp.float32),
        pltpu.VMEM((CH, O), jnp.float32),
        pltpu.VMEM_SHARED((N, O), jnp.float32),
        pltpu.SemaphoreType.DMA,
    ],
)(_sc_edges_body)


def _combine_body(p0_ref, p1_ref, base_ref, out_ref):
    out_ref[...] = p0_ref[...] + p1_ref[...] + base_ref[...]


_combine = pl.pallas_call(
    _combine_body,
    grid=(NB,),
    in_specs=[pl.BlockSpec((BN, O), lambda nb: (nb, 0))] * 3,
    out_specs=pl.BlockSpec((BN, O), lambda nb: (nb, 0)),
    out_shape=jax.ShapeDtypeStruct((N, O), jnp.float32),
)


def kernel(x, h, edge_index, edge_type, edge_weight, W_rel, W_root, bias,
           W_skip, b_skip):
    src = edge_index[0].reshape(E // 128, 128)
    et = edge_type.reshape(E // 128, 128)
    ew = edge_weight.reshape(NW, EPT)
    dst = edge_index[1].reshape(NW, NCH, CH)
    zeros = jnp.zeros((N, O), jnp.float32)

    idx = _idx(et, src).reshape(NW, EPT)
    hrel, base = _dense(h, x, W_rel, W_root, W_skip,
                        bias.reshape(1, O), b_skip.reshape(1, O))
    partials = _sc_edges(idx, ew, dst, hrel, zeros)
    return _combine(partials[0], partials[1], base)
